# baseline (speedup 1.0000x reference)
import functools

import jax
import jax.numpy as jnp
from jax import lax
from jax.experimental import pallas as pl
from jax.experimental.pallas import tpu as pltpu
from jax.experimental.pallas import tpu_sc as plsc

_NV = 8192
_D = 256
_NTOK = 8192
_BETA = 0.25
_M_BLK = 1024
_C_BLK = 2048
_NCHUNK = _NV // _C_BLK

_NW = 32
_B_PER_W = _NTOK // _NW
_CH = 128

_DN_T = (((1,), (1,)), ((), ()))


_HALF = _NTOK // 2
_KV_STRIP = _NV // (_HALF // _M_BLK)


def _argmin_half_body(with_kv, xi_ref, x_ref, kw_ref, ej_ref, v_ref,
                      idx_ref, *maybe_kv):
    xb = x_ref[...]
    xib = xi_ref[...]
    run_val = None
    run_idx = None
    for c in range(_NCHUNK):
        lo = c * _C_BLK
        s4 = lax.dot_general(xb, kw_ref[lo:lo + _C_BLK, :], _DN_T,
                             preferred_element_type=jnp.float32)
        t = xib + ej_ref[0:1, lo:lo + _C_BLK]
        d = t - s4
        bmin = jnp.min(d, axis=1, keepdims=True)
        ids = lax.broadcasted_iota(jnp.int32, d.shape, 1) + lo
        sel = jnp.where(d == bmin, ids, _NV)
        bidx = jnp.min(sel, axis=1, keepdims=True)
        if c == 0:
            run_val, run_idx = bmin, bidx
        else:
            upd = bmin < run_val
            run_idx = jnp.where(upd, bidx, run_idx)
            run_val = jnp.where(upd, bmin, run_val)
    rit = jnp.transpose(run_idx, (1, 0))
    for r in range(_M_BLK // 128):
        idx_ref[r:r + 1, :] = rit[:, r * 128:(r + 1) * 128]
    if with_kv:
        m = pl.program_id(0)
        maybe_kv[0][...] = lax.dot_general(
            kw_ref[pl.ds(pl.multiple_of(m * _KV_STRIP, _KV_STRIP),
                         _KV_STRIP), :],
            v_ref[...],
            (((1,), (0,)), ((), ())), preferred_element_type=jnp.float32)


def _argmin_half_call(xi2, x4_flat, kw, ej2row, v, off, with_kv):
    nsteps = _HALF // _M_BLK
    out_specs = [pl.BlockSpec((_M_BLK // 128, 128), lambda m: (m, 0))]
    out_shape = [jax.ShapeDtypeStruct((_HALF // 128, 128), jnp.int32)]
    if with_kv:
        out_specs.append(pl.BlockSpec((_KV_STRIP, _D), lambda m: (m, 0)))
        out_shape.append(jax.ShapeDtypeStruct((_NV, _D), jnp.float32))
    return pl.pallas_call(
        functools.partial(_argmin_half_body, with_kv),
        grid=(nsteps,),
        in_specs=[
            pl.BlockSpec((_M_BLK, 1), lambda m: (m + off, 0)),
            pl.BlockSpec((_M_BLK, _D), lambda m: (m + off, 0)),
            pl.BlockSpec((_NV, _D), lambda m: (0, 0)),
            pl.BlockSpec((1, _NV), lambda m: (0, 0)),
            pl.BlockSpec((_D, _D), lambda m: (0, 0)),
        ],
        out_specs=out_specs,
        out_shape=out_shape,
    )(xi2, x4_flat, kw, ej2row, v)


def _argmin_body(xi_ref, x_ref, kw_ref, ej_ref, v_ref, idx_ref, kv_ref):
    xb = x_ref[...]
    xib = xi_ref[...]
    run_val = None
    run_idx = None
    for c in range(_NCHUNK):
        lo = c * _C_BLK
        s4 = lax.dot_general(xb, kw_ref[lo:lo + _C_BLK, :], _DN_T,
                             preferred_element_type=jnp.float32)
        t = xib + ej_ref[0:1, lo:lo + _C_BLK]
        d = t - s4
        bmin = jnp.min(d, axis=1, keepdims=True)
        ids = lax.broadcasted_iota(jnp.int32, d.shape, 1) + lo
        sel = jnp.where(d == bmin, ids, _NV)
        bidx = jnp.min(sel, axis=1, keepdims=True)
        if c == 0:
            run_val, run_idx = bmin, bidx
        else:
            upd = bmin < run_val
            run_idx = jnp.where(upd, bidx, run_idx)
            run_val = jnp.where(upd, bmin, run_val)
    rit = jnp.transpose(run_idx, (1, 0))
    for r in range(_M_BLK // 128):
        idx_ref[r:r + 1, :] = rit[:, r * 128:(r + 1) * 128]
    m = pl.program_id(0)
    kv_ref[...] = lax.dot_general(
        kw_ref[pl.ds(pl.multiple_of(m * _M_BLK, _M_BLK), _M_BLK), :],
        v_ref[...],
        (((1,), (0,)), ((), ())), preferred_element_type=jnp.float32)


def _argmin_call(xi2, x4_flat, kw, ej2row, v):
    nsteps = _NTOK // _M_BLK
    nrow = _M_BLK // 128
    return pl.pallas_call(
        _argmin_body,
        grid=(nsteps,),
        in_specs=[
            pl.BlockSpec((_M_BLK, 1), lambda m: (m, 0)),
            pl.BlockSpec((_M_BLK, _D), lambda m: (m, 0)),
            pl.BlockSpec((_NV, _D), lambda m: (0, 0)),
            pl.BlockSpec((1, _NV), lambda m: (0, 0)),
            pl.BlockSpec((_D, _D), lambda m: (0, 0)),
        ],
        out_specs=[
            pl.BlockSpec((nrow, 128), lambda m: (m, 0)),
            pl.BlockSpec((_M_BLK, _D), lambda m: (m, 0)),
        ],
        out_shape=[
            jax.ShapeDtypeStruct((_NTOK // 128, 128), jnp.int32),
            jax.ShapeDtypeStruct((_NV, _D), jnp.float32),
        ],
    )(xi2, x4_flat, kw, ej2row, v)


def _sc_gather(table, idx, ntok=_NTOK):
    mesh = plsc.VectorSubcoreMesh(core_axis_name="c", subcore_axis_name="s")
    b_per_w = ntok // _NW

    @functools.partial(
        pl.kernel,
        mesh=mesh,
        out_type=jax.ShapeDtypeStruct((ntok, _D), jnp.float32),
        scratch_types=[
            pltpu.VMEM((_CH,), jnp.int32),
            pltpu.VMEM((_CH, _D), jnp.float32),
            pltpu.SemaphoreType.DMA,
        ],
    )
    def _g(table_hbm, idx_hbm, out_hbm, idx_v, rows_v, sem):
        wid = lax.axis_index("s") * 2 + lax.axis_index("c")
        base = wid * b_per_w
        for ci in range(b_per_w // _CH):
            off = base + ci * _CH
            pltpu.sync_copy(idx_hbm.at[pl.ds(off, _CH)], idx_v)
            pltpu.async_copy(table_hbm.at[idx_v], rows_v, sem).wait()
            pltpu.sync_copy(rows_v, out_hbm.at[pl.ds(off, _CH)])

    return _g(table, idx)


_E_BLK = 1024


def _epilogue_body(xq_ref, x4_ref, st_ref, grad_ref, loss_ref):
    i = pl.program_id(0)
    xq = xq_ref[...]
    xo = 0.25 * x4_ref[...]
    grad_ref[...] = xq
    st_ref[...] = xo + (xq - xo)
    part = jnp.sum((xq - xo) ** 2)

    @pl.when(i == 0)
    def _init():
        loss_ref[...] = jnp.zeros((1, 1), jnp.float32)

    loss_ref[...] = loss_ref[...] + part

    @pl.when(i == pl.num_programs(0) - 1)
    def _fin():
        m = loss_ref[...] / (_NTOK * _D)
        loss_ref[...] = m + _BETA * m


def _epilogue_call(xq, x4_flat):
    nb = _NTOK // _E_BLK
    return pl.pallas_call(
        _epilogue_body,
        grid=(nb,),
        in_specs=[
            pl.BlockSpec((_E_BLK, _D), lambda i: (i, 0)),
            pl.BlockSpec((_E_BLK, _D), lambda i: (i, 0)),
        ],
        out_specs=[
            pl.BlockSpec((_E_BLK, _D), lambda i: (i, 0)),
            pl.BlockSpec((_E_BLK, _D), lambda i: (i, 0)),
            pl.BlockSpec((1, 1), lambda i: (0, 0)),
        ],
        out_shape=[
            jax.ShapeDtypeStruct((_NTOK, _D), jnp.float32),
            jax.ShapeDtypeStruct((_NTOK, _D), jnp.float32),
            jax.ShapeDtypeStruct((1, 1), jnp.float32),
        ],
    )(xq, x4_flat)


def _epi_half_body(first, xq_ref, x4_ref, lin_ref, *refs):
    st_ref, grad_ref, loss_ref = refs[-3:]
    i = pl.program_id(0)
    xq = xq_ref[...]
    xo = 0.25 * x4_ref[...]
    grad_ref[...] = xq
    st_ref[...] = xo + (xq - xo)
    part = jnp.sum((xq - xo) ** 2)

    @pl.when(i == 0)
    def _init():
        loss_ref[...] = jnp.zeros((1, 1), jnp.float32) if first \
            else lin_ref[...]

    loss_ref[...] = loss_ref[...] + part

    if not first:
        @pl.when(i == pl.num_programs(0) - 1)
        def _fin():
            m = loss_ref[...] / (_NTOK * _D)
            loss_ref[...] = m + _BETA * m


def _epi_half_call(xq_h, x4_flat, off, first, st_prev, grad_prev, lin):
    nsteps = _HALF // _E_BLK
    in_specs = [
        pl.BlockSpec((_E_BLK, _D), lambda i: (i, 0)),
        pl.BlockSpec((_E_BLK, _D), lambda i: (i + off, 0)),
        pl.BlockSpec((1, 1), lambda i: (0, 0)),
    ]
    operands = [xq_h, x4_flat, lin]
    aliases = {}
    if not first:
        in_specs += [pl.BlockSpec(memory_space=pl.ANY),
                     pl.BlockSpec(memory_space=pl.ANY)]
        operands += [st_prev, grad_prev]
        aliases = {3: 0, 4: 1}
    return pl.pallas_call(
        functools.partial(_epi_half_body, first),
        grid=(nsteps,),
        in_specs=in_specs,
        out_specs=[
            pl.BlockSpec((_E_BLK, _D), lambda i: (i + off, 0)),
            pl.BlockSpec((_E_BLK, _D), lambda i: (i + off, 0)),
            pl.BlockSpec((1, 1), lambda i: (0, 0)),
        ],
        out_shape=[
            jax.ShapeDtypeStruct((_NTOK, _D), jnp.float32),
            jax.ShapeDtypeStruct((_NTOK, _D), jnp.float32),
            jax.ShapeDtypeStruct((1, 1), jnp.float32),
        ],
        input_output_aliases=aliases,
    )(*operands)


def kernel(x, key_weight, value_weight):
    b, c, h, w = x.shape
    x_t = jnp.transpose(x, (0, 2, 3, 1))
    x4_flat = 4.0 * x_t.reshape(-1, c)
    xi2 = 2.0 * jnp.sum((0.25 * x4_flat) ** 2, axis=1, keepdims=True)
    ej2row = (2.0 * jnp.sum(key_weight ** 2, axis=1))[None, :]
    args = (xi2, x4_flat, key_weight, ej2row, value_weight)
    half_steps = _HALF // _M_BLK
    i1, kv = _argmin_half_call(*args, 0, True)
    (i2,) = _argmin_half_call(*args, half_steps, False)
    xq1 = _sc_gather(kv, i1.reshape(_HALF), _HALF)
    xq2 = _sc_gather(kv, i2.reshape(_HALF), _HALF)
    zero = jnp.zeros((1, 1), jnp.float32)
    st1, gr1, ls1 = _epi_half_call(xq1, x4_flat, 0, True, None, None, zero)
    st, grad, loss11 = _epi_half_call(
        xq2, x4_flat, half_steps, False, st1, gr1, ls1)
    idx = jnp.concatenate([i1.reshape(_HALF), i2.reshape(_HALF)])
    y_st = jnp.transpose(st.reshape(b, h, w, c), (0, 3, 1, 2))
    y_grad = jnp.transpose(grad.reshape(b, h, w, c), (0, 3, 1, 2))
    return (y_st, y_grad, idx, loss11[0, 0])


# speedup vs baseline: 1.0285x; 1.0285x over previous
import functools

import jax
import jax.numpy as jnp
from jax import lax
from jax.experimental import pallas as pl
from jax.experimental.pallas import tpu as pltpu
from jax.experimental.pallas import tpu_sc as plsc

_NV = 8192
_D = 256
_NTOK = 8192
_BETA = 0.25
_M_BLK = 1024
_C_BLK = 2048
_NCHUNK = _NV // _C_BLK

_NW = 32
_B_PER_W = _NTOK // _NW
_CH = 128

_DN_T = (((1,), (1,)), ((), ()))


def _argmin_body(xi_ref, x_ref, kw_ref, ej_ref, v_ref, idx_ref, kv_ref):
    xb = x_ref[...]
    xib = xi_ref[...]
    run_val = None
    run_idx = None
    for c in range(_NCHUNK):
        lo = c * _C_BLK
        s4 = lax.dot_general(xb, kw_ref[lo:lo + _C_BLK, :], _DN_T,
                             preferred_element_type=jnp.float32)
        t = xib + ej_ref[0:1, lo:lo + _C_BLK]
        d = t - s4
        bmin = jnp.min(d, axis=1, keepdims=True)
        ids = lax.broadcasted_iota(jnp.int32, d.shape, 1) + lo
        sel = jnp.where(d == bmin, ids, _NV)
        bidx = jnp.min(sel, axis=1, keepdims=True)
        if c == 0:
            run_val, run_idx = bmin, bidx
        else:
            upd = bmin < run_val
            run_idx = jnp.where(upd, bidx, run_idx)
            run_val = jnp.where(upd, bmin, run_val)
    rit = jnp.transpose(run_idx, (1, 0))
    for r in range(_M_BLK // 128):
        idx_ref[r:r + 1, :] = rit[:, r * 128:(r + 1) * 128]
    m = pl.program_id(0)
    kv_ref[...] = lax.dot_general(
        kw_ref[pl.ds(pl.multiple_of(m * _M_BLK, _M_BLK), _M_BLK), :],
        v_ref[...],
        (((1,), (0,)), ((), ())), preferred_element_type=jnp.float32)


def _argmin_call(xi2, x4_flat, kw, ej2row, v):
    nsteps = _NTOK // _M_BLK
    nrow = _M_BLK // 128
    return pl.pallas_call(
        _argmin_body,
        grid=(nsteps,),
        in_specs=[
            pl.BlockSpec((_M_BLK, 1), lambda m: (m, 0)),
            pl.BlockSpec((_M_BLK, _D), lambda m: (m, 0)),
            pl.BlockSpec((_NV, _D), lambda m: (0, 0)),
            pl.BlockSpec((1, _NV), lambda m: (0, 0)),
            pl.BlockSpec((_D, _D), lambda m: (0, 0)),
        ],
        out_specs=[
            pl.BlockSpec((nrow, 128), lambda m: (m, 0)),
            pl.BlockSpec((_M_BLK, _D), lambda m: (m, 0)),
        ],
        out_shape=[
            jax.ShapeDtypeStruct((_NTOK // 128, 128), jnp.int32),
            jax.ShapeDtypeStruct((_NV, _D), jnp.float32),
        ],
    )(xi2, x4_flat, kw, ej2row, v)


def _sc_gather(table, idx):
    mesh = plsc.VectorSubcoreMesh(core_axis_name="c", subcore_axis_name="s")

    @functools.partial(
        pl.kernel,
        mesh=mesh,
        out_type=jax.ShapeDtypeStruct((_NTOK, _D), jnp.float32),
        scratch_types=[
            pltpu.VMEM((_CH,), jnp.int32),
            pltpu.VMEM((_CH, _D), jnp.float32),
            pltpu.SemaphoreType.DMA,
        ],
    )
    def _g(table_hbm, idx_hbm, out_hbm, idx_v, rows_v, sem):
        wid = lax.axis_index("s") * 2 + lax.axis_index("c")
        base = wid * _B_PER_W
        for ci in range(_B_PER_W // _CH):
            off = base + ci * _CH
            pltpu.sync_copy(idx_hbm.at[pl.ds(off, _CH)], idx_v)
            pltpu.async_copy(table_hbm.at[idx_v], rows_v, sem).wait()
            pltpu.sync_copy(rows_v, out_hbm.at[pl.ds(off, _CH)])

    return _g(table, idx)


_E_BLK = 1024


def _epilogue_body(xq_ref, x4_ref, st_ref, grad_ref, loss_ref):
    i = pl.program_id(0)
    xq = xq_ref[...]
    xo = 0.25 * x4_ref[...]
    grad_ref[...] = xq
    st_ref[...] = xo + (xq - xo)
    part = jnp.sum((xq - xo) ** 2)

    @pl.when(i == 0)
    def _init():
        loss_ref[...] = jnp.zeros((1, 1), jnp.float32)

    loss_ref[...] = loss_ref[...] + part

    @pl.when(i == pl.num_programs(0) - 1)
    def _fin():
        m = loss_ref[...] / (_NTOK * _D)
        loss_ref[...] = m + _BETA * m


def _epilogue_call(xq, x4_flat):
    nb = _NTOK // _E_BLK
    return pl.pallas_call(
        _epilogue_body,
        grid=(nb,),
        in_specs=[
            pl.BlockSpec((_E_BLK, _D), lambda i: (i, 0)),
            pl.BlockSpec((_E_BLK, _D), lambda i: (i, 0)),
        ],
        out_specs=[
            pl.BlockSpec((_E_BLK, _D), lambda i: (i, 0)),
            pl.BlockSpec((_E_BLK, _D), lambda i: (i, 0)),
            pl.BlockSpec((1, 1), lambda i: (0, 0)),
        ],
        out_shape=[
            jax.ShapeDtypeStruct((_NTOK, _D), jnp.float32),
            jax.ShapeDtypeStruct((_NTOK, _D), jnp.float32),
            jax.ShapeDtypeStruct((1, 1), jnp.float32),
        ],
    )(xq, x4_flat)


def kernel(x, key_weight, value_weight):
    b, c, h, w = x.shape
    x_t = jnp.transpose(x, (0, 2, 3, 1))
    x4_flat = 4.0 * x_t.reshape(-1, c)
    xi2 = 2.0 * jnp.sum((0.25 * x4_flat) ** 2, axis=1, keepdims=True)
    ej2row = (2.0 * jnp.sum(key_weight ** 2, axis=1))[None, :]
    idx2, kv = _argmin_call(xi2, x4_flat, key_weight, ej2row, value_weight)
    idx = idx2.reshape(_NTOK)
    xq = _sc_gather(kv, idx)
    st, grad, loss11 = _epilogue_call(xq, x4_flat)
    y_st = jnp.transpose(st.reshape(b, h, w, c), (0, 3, 1, 2))
    y_grad = jnp.transpose(grad.reshape(b, h, w, c), (0, 3, 1, 2))
    return (y_st, y_grad, idx, loss11[0, 0])
